# baseline (device time: 1622879 ns/iter reference)
import jax
import jax.numpy as jnp
from jax import lax
from jax.experimental import pallas as pl
from jax.experimental.pallas import tpu as pltpu

N_DEV = 32


def kernel(x, w_mat, scale_x, scale_w):
    m_total, k_per = x.shape
    k_per_w, n = w_mat.shape
    assert k_per == k_per_w
    assert m_total % N_DEV == 0
    m_per = m_total // N_DEV

    def body(x_ref, w_ref, sx_ref, sw_ref, out_ref,
             comm_ref, pbuf_ref, xb_ref, wb_ref,
             send_sems, recv_sems, credit_sem):
        my = lax.axis_index("i")
        left = lax.rem(my + N_DEV - 1, N_DEV)
        right = lax.rem(my + 1, N_DEV)

        barrier_sem = pltpu.get_barrier_semaphore()
        for nbr in (left, right):
            pl.semaphore_signal(barrier_sem, inc=1, device_id=(nbr,),
                                device_id_type=pl.DeviceIdType.MESH)
        pl.semaphore_wait(barrier_sem, 2)

        xb_ref[...] = x_ref[...].astype(jnp.bfloat16)
        wb_ref[...] = w_ref[...].astype(jnp.bfloat16)

        def partial_for(chunk):
            rows = xb_ref[pl.ds(chunk * m_per, m_per), :]
            return jnp.dot(rows, wb_ref[...],
                           preferred_element_type=jnp.float32)

        comm_ref[0] = partial_for(lax.rem(my + N_DEV - 1, N_DEV))

        for h in range(N_DEV - 1):
            ss = h % 2
            rs = (h + 1) % 2
            if h >= 1:
                pl.semaphore_wait(credit_sem, 1)
            rdma = pltpu.make_async_remote_copy(
                src_ref=comm_ref.at[ss],
                dst_ref=comm_ref.at[rs],
                send_sem=send_sems.at[ss],
                recv_sem=recv_sems.at[rs],
                device_id=(right,),
                device_id_type=pl.DeviceIdType.MESH,
            )
            rdma.start()
            chunk = lax.rem(my + 2 * N_DEV - h - 2, N_DEV)
            pbuf_ref[...] = partial_for(chunk)
            rdma.wait()
            if h < N_DEV - 2:
                pl.semaphore_signal(credit_sem, inc=1, device_id=(left,),
                                    device_id_type=pl.DeviceIdType.MESH)
                comm_ref[rs] = comm_ref[rs] + pbuf_ref[...]
            else:
                total = comm_ref[rs] + pbuf_ref[...]
                scale = sx_ref[0] * sw_ref[0]
                out_ref[...] = jnp.maximum(total * scale, 0.0)

    return pl.pallas_call(
        body,
        out_shape=jax.ShapeDtypeStruct((m_per, n), jnp.float32),
        in_specs=[
            pl.BlockSpec(memory_space=pltpu.VMEM),
            pl.BlockSpec(memory_space=pltpu.VMEM),
            pl.BlockSpec(memory_space=pltpu.SMEM),
            pl.BlockSpec(memory_space=pltpu.SMEM),
        ],
        out_specs=pl.BlockSpec(memory_space=pltpu.VMEM),
        scratch_shapes=[
            pltpu.VMEM((2, m_per, n), jnp.float32),
            pltpu.VMEM((m_per, n), jnp.float32),
            pltpu.VMEM((m_total, k_per), jnp.bfloat16),
            pltpu.VMEM((k_per, n), jnp.bfloat16),
            pltpu.SemaphoreType.DMA((2,)),
            pltpu.SemaphoreType.DMA((2,)),
            pltpu.SemaphoreType.REGULAR,
        ],
        compiler_params=pltpu.CompilerParams(collective_id=0),
    )(x, w_mat, scale_x, scale_w)


# device time: 779216 ns/iter; 2.0827x vs baseline; 2.0827x over previous
import jax
import jax.numpy as jnp
from jax import lax
from jax.experimental import pallas as pl
from jax.experimental.pallas import tpu as pltpu

N_DEV = 32


def kernel(x, w_mat, scale_x, scale_w):
    m_total, k_per = x.shape
    k_per_w, n = w_mat.shape
    assert k_per == k_per_w
    assert m_total % N_DEV == 0 and n % 2 == 0
    m_per = m_total // N_DEV
    n_half = n // 2

    def body(x_ref, w_ref, sx_ref, sw_ref, out_ref,
             comm_r_ref, comm_l_ref, pbuf_r_ref, pbuf_l_ref,
             xb_ref, wb_ref,
             send_r, recv_r, send_l, recv_l, credit_r, credit_l):
        my = lax.axis_index("i")
        left = lax.rem(my + N_DEV - 1, N_DEV)
        right = lax.rem(my + 1, N_DEV)

        barrier_sem = pltpu.get_barrier_semaphore()
        for nbr in (left, right):
            pl.semaphore_signal(barrier_sem, inc=1, device_id=(nbr,),
                                device_id_type=pl.DeviceIdType.MESH)
        pl.semaphore_wait(barrier_sem, 2)

        xb_ref[...] = x_ref[...].astype(jnp.bfloat16)
        wb_ref[...] = w_ref[...].astype(jnp.bfloat16)

        def partial_for(chunk, col0):
            rows = xb_ref[pl.ds(chunk * m_per, m_per), :]
            return jnp.dot(rows, wb_ref[:, col0:col0 + n_half],
                           preferred_element_type=jnp.float32)

        comm_r_ref[0] = partial_for(
            lax.rem(my + N_DEV - 1, N_DEV), 0).astype(jnp.bfloat16)
        comm_l_ref[0] = partial_for(
            lax.rem(my + 1, N_DEV), n_half).astype(jnp.bfloat16)

        for h in range(N_DEV - 1):
            ss = h % 2
            rs = (h + 1) % 2
            if h >= 1:
                pl.semaphore_wait(credit_r, 1)
                pl.semaphore_wait(credit_l, 1)
            rdma_r = pltpu.make_async_remote_copy(
                src_ref=comm_r_ref.at[ss], dst_ref=comm_r_ref.at[rs],
                send_sem=send_r.at[ss], recv_sem=recv_r.at[rs],
                device_id=(right,), device_id_type=pl.DeviceIdType.MESH,
            )
            rdma_l = pltpu.make_async_remote_copy(
                src_ref=comm_l_ref.at[ss], dst_ref=comm_l_ref.at[rs],
                send_sem=send_l.at[ss], recv_sem=recv_l.at[rs],
                device_id=(left,), device_id_type=pl.DeviceIdType.MESH,
            )
            rdma_r.start()
            rdma_l.start()
            chunk_r = lax.rem(my + 2 * N_DEV - h - 2, N_DEV)
            chunk_l = lax.rem(my + h + 2, N_DEV)
            pbuf_r_ref[...] = partial_for(chunk_r, 0)
            pbuf_l_ref[...] = partial_for(chunk_l, n_half)
            rdma_r.wait()
            rdma_l.wait()
            if h < N_DEV - 2:
                pl.semaphore_signal(credit_r, inc=1, device_id=(left,),
                                    device_id_type=pl.DeviceIdType.MESH)
                pl.semaphore_signal(credit_l, inc=1, device_id=(right,),
                                    device_id_type=pl.DeviceIdType.MESH)
                comm_r_ref[rs] = (comm_r_ref[rs].astype(jnp.float32)
                                  + pbuf_r_ref[...]).astype(jnp.bfloat16)
                comm_l_ref[rs] = (comm_l_ref[rs].astype(jnp.float32)
                                  + pbuf_l_ref[...]).astype(jnp.bfloat16)
            else:
                scale = sx_ref[0] * sw_ref[0]
                tot_r = comm_r_ref[rs].astype(jnp.float32) + pbuf_r_ref[...]
                tot_l = comm_l_ref[rs].astype(jnp.float32) + pbuf_l_ref[...]
                out_ref[:, 0:n_half] = jnp.maximum(tot_r * scale, 0.0)
                out_ref[:, n_half:n] = jnp.maximum(tot_l * scale, 0.0)

    return pl.pallas_call(
        body,
        out_shape=jax.ShapeDtypeStruct((m_per, n), jnp.float32),
        in_specs=[
            pl.BlockSpec(memory_space=pltpu.VMEM),
            pl.BlockSpec(memory_space=pltpu.VMEM),
            pl.BlockSpec(memory_space=pltpu.SMEM),
            pl.BlockSpec(memory_space=pltpu.SMEM),
        ],
        out_specs=pl.BlockSpec(memory_space=pltpu.VMEM),
        scratch_shapes=[
            pltpu.VMEM((2, m_per, n_half), jnp.bfloat16),
            pltpu.VMEM((2, m_per, n_half), jnp.bfloat16),
            pltpu.VMEM((m_per, n_half), jnp.float32),
            pltpu.VMEM((m_per, n_half), jnp.float32),
            pltpu.VMEM((m_total, k_per), jnp.bfloat16),
            pltpu.VMEM((k_per, n), jnp.bfloat16),
            pltpu.SemaphoreType.DMA((2,)),
            pltpu.SemaphoreType.DMA((2,)),
            pltpu.SemaphoreType.DMA((2,)),
            pltpu.SemaphoreType.DMA((2,)),
            pltpu.SemaphoreType.REGULAR,
            pltpu.SemaphoreType.REGULAR,
        ],
        compiler_params=pltpu.CompilerParams(collective_id=0),
    )(x, w_mat, scale_x, scale_w)


# device time: 441316 ns/iter; 3.6774x vs baseline; 1.7657x over previous
import numpy as np

import jax
import jax.numpy as jnp
from jax import lax
from jax.experimental import pallas as pl
from jax.experimental.pallas import tpu as pltpu

N_DEV = 32


def _hamiltonian_ring() -> np.ndarray:

    def lid(x: int, y: int, z: int) -> int:
        return z * 8 + y * 2 + (x if y % 2 == 0 else 1 - x)

    cycle: list[tuple[int, int, int]] = []
    for z in range(4):
        ys = range(4) if z % 2 == 0 else range(3, -1, -1)
        cycle.extend((0, y, z) for y in ys)
    for z in range(3, -1, -1):
        ys = range(4) if z % 2 == 1 else range(3, -1, -1)
        cycle.extend((1, y, z) for y in ys)
    for p in range(32):
        a, b = cycle[p], cycle[(p + 1) % 32]
        assert sum(abs(a[i] - b[i]) for i in range(3)) == 1, (p, a, b)
    return np.array([lid(*c) for c in cycle], dtype=np.int32)


_RING = _hamiltonian_ring()
_POS = np.argsort(_RING).astype(np.int32)


def kernel(x, w_mat, scale_x, scale_w):
    m_total, k_per = x.shape
    k_per_w, n = w_mat.shape
    assert k_per == k_per_w
    assert m_total % N_DEV == 0 and n % 2 == 0
    m_per = m_total // N_DEV
    n_half = n // 2

    ring = jnp.asarray(_RING)
    pos = jnp.asarray(_POS)
    my = lax.axis_index("i")
    rho = pos[my]
    nbrs = jnp.stack([ring[(rho + 1) % N_DEV],
                      ring[(rho - 1) % N_DEV]]).astype(jnp.int32)
    js = jnp.arange(N_DEV, dtype=jnp.int32)
    tab_r = ring[(rho - 1 - js) % N_DEV].astype(jnp.int32)
    tab_l = ring[(rho + 1 + js) % N_DEV].astype(jnp.int32)

    def body(x_ref, w_ref, sx_ref, sw_ref, nbr_ref, tr_ref, tl_ref,
             out_ref, comm_r_ref, comm_l_ref, pbuf_r_ref, pbuf_l_ref,
             xb_ref, wb_ref,
             send_r, recv_r, send_l, recv_l, credit_r, credit_l):
        right = nbr_ref[0]
        left = nbr_ref[1]

        barrier_sem = pltpu.get_barrier_semaphore()
        for nbr in (left, right):
            pl.semaphore_signal(barrier_sem, inc=1, device_id=(nbr,),
                                device_id_type=pl.DeviceIdType.MESH)
        pl.semaphore_wait(barrier_sem, 2)

        xb_ref[...] = x_ref[...].astype(jnp.bfloat16)
        wb_ref[...] = w_ref[...].astype(jnp.bfloat16)

        def partial_for(chunk, col0):
            rows = xb_ref[pl.ds(chunk * m_per, m_per), :]
            return jnp.dot(rows, wb_ref[:, col0:col0 + n_half],
                           preferred_element_type=jnp.float32)

        comm_r_ref[0] = partial_for(tr_ref[0], 0).astype(jnp.bfloat16)
        comm_l_ref[0] = partial_for(tl_ref[0], n_half).astype(jnp.bfloat16)

        for h in range(N_DEV - 1):
            ss = h % 2
            rs = (h + 1) % 2
            if h >= 1:
                pl.semaphore_wait(credit_r, 1)
                pl.semaphore_wait(credit_l, 1)
            rdma_r = pltpu.make_async_remote_copy(
                src_ref=comm_r_ref.at[ss], dst_ref=comm_r_ref.at[rs],
                send_sem=send_r.at[ss], recv_sem=recv_r.at[rs],
                device_id=(right,), device_id_type=pl.DeviceIdType.MESH,
            )
            rdma_l = pltpu.make_async_remote_copy(
                src_ref=comm_l_ref.at[ss], dst_ref=comm_l_ref.at[rs],
                send_sem=send_l.at[ss], recv_sem=recv_l.at[rs],
                device_id=(left,), device_id_type=pl.DeviceIdType.MESH,
            )
            rdma_r.start()
            rdma_l.start()
            pbuf_r_ref[...] = partial_for(tr_ref[h + 1], 0)
            pbuf_l_ref[...] = partial_for(tl_ref[h + 1], n_half)
            rdma_r.wait()
            rdma_l.wait()
            if h < N_DEV - 2:
                pl.semaphore_signal(credit_r, inc=1, device_id=(left,),
                                    device_id_type=pl.DeviceIdType.MESH)
                pl.semaphore_signal(credit_l, inc=1, device_id=(right,),
                                    device_id_type=pl.DeviceIdType.MESH)
                comm_r_ref[rs] = (comm_r_ref[rs].astype(jnp.float32)
                                  + pbuf_r_ref[...]).astype(jnp.bfloat16)
                comm_l_ref[rs] = (comm_l_ref[rs].astype(jnp.float32)
                                  + pbuf_l_ref[...]).astype(jnp.bfloat16)
            else:
                scale = sx_ref[0] * sw_ref[0]
                tot_r = comm_r_ref[rs].astype(jnp.float32) + pbuf_r_ref[...]
                tot_l = comm_l_ref[rs].astype(jnp.float32) + pbuf_l_ref[...]
                out_ref[:, 0:n_half] = jnp.maximum(tot_r * scale, 0.0)
                out_ref[:, n_half:n] = jnp.maximum(tot_l * scale, 0.0)

    return pl.pallas_call(
        body,
        out_shape=jax.ShapeDtypeStruct((m_per, n), jnp.float32),
        in_specs=[
            pl.BlockSpec(memory_space=pltpu.VMEM),
            pl.BlockSpec(memory_space=pltpu.VMEM),
            pl.BlockSpec(memory_space=pltpu.SMEM),
            pl.BlockSpec(memory_space=pltpu.SMEM),
            pl.BlockSpec(memory_space=pltpu.SMEM),
            pl.BlockSpec(memory_space=pltpu.SMEM),
            pl.BlockSpec(memory_space=pltpu.SMEM),
        ],
        out_specs=pl.BlockSpec(memory_space=pltpu.VMEM),
        scratch_shapes=[
            pltpu.VMEM((2, m_per, n_half), jnp.bfloat16),
            pltpu.VMEM((2, m_per, n_half), jnp.bfloat16),
            pltpu.VMEM((m_per, n_half), jnp.float32),
            pltpu.VMEM((m_per, n_half), jnp.float32),
            pltpu.VMEM((m_total, k_per), jnp.bfloat16),
            pltpu.VMEM((k_per, n), jnp.bfloat16),
            pltpu.SemaphoreType.DMA((2,)),
            pltpu.SemaphoreType.DMA((2,)),
            pltpu.SemaphoreType.DMA((2,)),
            pltpu.SemaphoreType.DMA((2,)),
            pltpu.SemaphoreType.REGULAR,
            pltpu.SemaphoreType.REGULAR,
        ],
        compiler_params=pltpu.CompilerParams(collective_id=0),
    )(x, w_mat, scale_x, scale_w, nbrs, tab_r, tab_l)


# device time: 365874 ns/iter; 4.4356x vs baseline; 1.2062x over previous
import numpy as np

import jax
import jax.numpy as jnp
from jax import lax
from jax.experimental import pallas as pl
from jax.experimental.pallas import tpu as pltpu

N_DEV = 32


def _hamiltonian_ring() -> np.ndarray:

    def lid(x: int, y: int, z: int) -> int:
        return z * 8 + y * 2 + (x if y % 2 == 0 else 1 - x)

    cycle: list[tuple[int, int, int]] = []
    for z in range(4):
        ys = range(4) if z % 2 == 0 else range(3, -1, -1)
        cycle.extend((0, y, z) for y in ys)
    for z in range(3, -1, -1):
        ys = range(4) if z % 2 == 1 else range(3, -1, -1)
        cycle.extend((1, y, z) for y in ys)
    for p in range(32):
        a, b = cycle[p], cycle[(p + 1) % 32]
        assert sum(abs(a[i] - b[i]) for i in range(3)) == 1, (p, a, b)
    return np.array([lid(*c) for c in cycle], dtype=np.int32)


_RING = _hamiltonian_ring()
_POS = np.argsort(_RING).astype(np.int32)


def kernel(x, w_mat, scale_x, scale_w):
    m_total, k_per = x.shape
    k_per_w, n = w_mat.shape
    assert k_per == k_per_w
    assert m_total % N_DEV == 0 and n % 4 == 0
    m_per = m_total // N_DEV
    n_half = n // 2
    n_q = n // 4

    ring = jnp.asarray(_RING)
    pos = jnp.asarray(_POS)
    my = lax.axis_index("i")
    rho = pos[my]
    nbrs = jnp.stack([ring[(rho + 1) % N_DEV],
                      ring[(rho - 1) % N_DEV]]).astype(jnp.int32)
    js = jnp.arange(N_DEV, dtype=jnp.int32)
    tab_r = ring[(rho - 1 - js) % N_DEV].astype(jnp.int32)
    tab_l = ring[(rho + 1 + js) % N_DEV].astype(jnp.int32)

    def body(x_ref, w_ref, sx_ref, sw_ref, nbr_ref, tr_ref, tl_ref,
             out_ref,
             c_r0, c_r1, c_l0, c_l1, pbuf_r_ref, pbuf_l_ref,
             xb_ref, wb_ref,
             s_r0, r_r0, s_r1, r_r1, s_l0, r_l0, s_l1, r_l1,
             k_r0, k_r1, k_l0, k_l1):
        right = nbr_ref[0]
        left = nbr_ref[1]

        barrier_sem = pltpu.get_barrier_semaphore()
        for nbr in (left, right):
            pl.semaphore_signal(barrier_sem, inc=1, device_id=(nbr,),
                                device_id_type=pl.DeviceIdType.MESH)
        pl.semaphore_wait(barrier_sem, 2)

        xb_ref[...] = x_ref[...].astype(jnp.bfloat16)
        wb_ref[...] = w_ref[...].astype(jnp.bfloat16)

        def partial_for(chunk, col0):
            rows = xb_ref[pl.ds(chunk * m_per, m_per), :]
            return jnp.dot(rows, wb_ref[:, col0:col0 + n_half],
                           preferred_element_type=jnp.float32)

        stripes = (
            (c_r0, s_r0, r_r0, k_r0, pbuf_r_ref, 0, right, left),
            (c_l0, s_l0, r_l0, k_l0, pbuf_l_ref, 0, left, right),
            (c_r1, s_r1, r_r1, k_r1, pbuf_r_ref, 1, right, left),
            (c_l1, s_l1, r_l1, k_l1, pbuf_l_ref, 1, left, right),
        )

        seed_r = partial_for(tr_ref[0], 0).astype(jnp.bfloat16)
        seed_l = partial_for(tl_ref[0], n_half).astype(jnp.bfloat16)
        c_r0[0] = seed_r[:, 0:n_q]
        c_r1[0] = seed_r[:, n_q:n_half]
        c_l0[0] = seed_l[:, 0:n_q]
        c_l1[0] = seed_l[:, n_q:n_half]

        def make(comm, ssem, rsem, dst, h):
            ss, rs = h % 2, (h + 1) % 2
            return pltpu.make_async_remote_copy(
                src_ref=comm.at[ss], dst_ref=comm.at[rs],
                send_sem=ssem.at[ss], recv_sem=rsem.at[rs],
                device_id=(dst,), device_id_type=pl.DeviceIdType.MESH,
            )

        inflight = []
        for comm, ssem, rsem, kred, pbuf, sub, dst, up in stripes:
            rdma = make(comm, ssem, rsem, dst, 0)
            rdma.start()
            inflight.append(rdma)

        pbuf_r_ref[...] = partial_for(tr_ref[1], 0)
        pbuf_l_ref[...] = partial_for(tl_ref[1], n_half)

        for h in range(1, N_DEV - 1):
            ss = h % 2
            nxt = []
            for i, (comm, ssem, rsem, kred, pbuf, sub, dst, up) \
                    in enumerate(stripes):
                prev = inflight[i]
                prev.wait_recv()
                comm[ss] = (comm[ss].astype(jnp.float32)
                            + pbuf[:, sub * n_q:(sub + 1) * n_q]
                            ).astype(jnp.bfloat16)
                prev.wait_send()
                pl.semaphore_signal(kred, inc=1, device_id=(up,),
                                    device_id_type=pl.DeviceIdType.MESH)
                pl.semaphore_wait(kred, 1)
                rdma = make(comm, ssem, rsem, dst, h)
                rdma.start()
                nxt.append(rdma)
            inflight = nxt
            pbuf_r_ref[...] = partial_for(tr_ref[h + 1], 0)
            pbuf_l_ref[...] = partial_for(tl_ref[h + 1], n_half)

        ss = (N_DEV - 1) % 2
        scale = sx_ref[0] * sw_ref[0]
        for i, (comm, ssem, rsem, kred, pbuf, sub, dst, up) \
                in enumerate(stripes):
            prev = inflight[i]
            prev.wait_recv()
            prev.wait_send()
            tot = (comm[ss].astype(jnp.float32)
                   + pbuf[:, sub * n_q:(sub + 1) * n_q])
            col0 = (0 if pbuf is pbuf_r_ref else n_half) + sub * n_q
            out_ref[:, col0:col0 + n_q] = jnp.maximum(tot * scale, 0.0)

    return pl.pallas_call(
        body,
        out_shape=jax.ShapeDtypeStruct((m_per, n), jnp.float32),
        in_specs=[
            pl.BlockSpec(memory_space=pltpu.VMEM),
            pl.BlockSpec(memory_space=pltpu.VMEM),
            pl.BlockSpec(memory_space=pltpu.SMEM),
            pl.BlockSpec(memory_space=pltpu.SMEM),
            pl.BlockSpec(memory_space=pltpu.SMEM),
            pl.BlockSpec(memory_space=pltpu.SMEM),
            pl.BlockSpec(memory_space=pltpu.SMEM),
        ],
        out_specs=pl.BlockSpec(memory_space=pltpu.VMEM),
        scratch_shapes=[
            pltpu.VMEM((2, m_per, n_q), jnp.bfloat16),
            pltpu.VMEM((2, m_per, n_q), jnp.bfloat16),
            pltpu.VMEM((2, m_per, n_q), jnp.bfloat16),
            pltpu.VMEM((2, m_per, n_q), jnp.bfloat16),
            pltpu.VMEM((m_per, n_half), jnp.float32),
            pltpu.VMEM((m_per, n_half), jnp.float32),
            pltpu.VMEM((m_total, k_per), jnp.bfloat16),
            pltpu.VMEM((k_per, n), jnp.bfloat16),
            pltpu.SemaphoreType.DMA((2,)),
            pltpu.SemaphoreType.DMA((2,)),
            pltpu.SemaphoreType.DMA((2,)),
            pltpu.SemaphoreType.DMA((2,)),
            pltpu.SemaphoreType.DMA((2,)),
            pltpu.SemaphoreType.DMA((2,)),
            pltpu.SemaphoreType.DMA((2,)),
            pltpu.SemaphoreType.DMA((2,)),
            pltpu.SemaphoreType.REGULAR,
            pltpu.SemaphoreType.REGULAR,
            pltpu.SemaphoreType.REGULAR,
            pltpu.SemaphoreType.REGULAR,
        ],
        compiler_params=pltpu.CompilerParams(collective_id=0),
    )(x, w_mat, scale_x, scale_w, nbrs, tab_r, tab_l)
